# baseline (speedup 1.0000x reference)
import jax
import jax.numpy as jnp
from jax.experimental import pallas as pl
from jax.experimental.pallas import tpu as pltpu

QB = 1024
KB = 2048


def _matmul_body(q_ref, k_ref, out_ref):
    out_ref[...] = jax.lax.dot_general(
        q_ref[...], k_ref[...],
        dimension_numbers=(((1,), (1,)), ((), ())),
        preferred_element_type=jnp.float32,
    )


def kernel(queries, keys, k):
    n = keys.shape[0]
    n_pad = ((n + KB - 1) // KB) * KB
    keys_p = jnp.pad(keys, ((0, n_pad - n), (0, 0)))
    grid = (n_pad // KB,)
    logits = pl.pallas_call(
        _matmul_body,
        grid=grid,
        in_specs=[
            pl.BlockSpec((QB, 128), lambda j: (0, 0)),
            pl.BlockSpec((KB, 128), lambda j: (j, 0)),
        ],
        out_specs=pl.BlockSpec((QB, KB), lambda j: (0, j)),
        out_shape=jax.ShapeDtypeStruct((QB, n_pad), jnp.float32),
    )(queries, keys_p)
    values, indices = jax.lax.top_k(logits[:, :n], 100)
    return values, indices


# speedup vs baseline: 8.0999x; 8.0999x over previous
import functools

import jax
import jax.numpy as jnp
from jax import lax
from jax.experimental import pallas as pl
from jax.experimental.pallas import tpu as pltpu
from jax.experimental.pallas import tpu_sc as plsc

NQ = 1024
NKEYS = 100000
CHUNK = 128
NCHUNK = 784
NPAD = NCHUNK * CHUNK
KB = 2048
TOPK = 100
NSEL = TOPK * CHUNK
NEG = -3.0e38
BIGI = 2**30

def _mm_body(q_ref, k_ref, out_ref, m_ref):
    j = pl.program_id(0)
    s = lax.dot_general(q_ref[...], k_ref[...],
                        dimension_numbers=(((1,), (1,)), ((), ())),
                        preferred_element_type=jnp.float32)
    col = j * KB + lax.broadcasted_iota(jnp.int32, s.shape, 1)
    s = jnp.where(col < NKEYS, s, NEG)
    out_ref[...] = s
    m = jnp.max(s.reshape(NQ, KB // CHUNK, CHUNK), axis=2)
    m_ref[...] = m.reshape(1, NQ, KB // CHUNK)


def _matmul(queries, keys_p):
    return pl.pallas_call(
        _mm_body,
        grid=(NPAD // KB,),
        in_specs=[
            pl.BlockSpec((NQ, 128), lambda j: (0, 0)),
            pl.BlockSpec((KB, 128), lambda j: (j, 0)),
        ],
        out_specs=[
            pl.BlockSpec((NQ, KB), lambda j: (0, j)),
            pl.BlockSpec((1, NQ, KB // CHUNK), lambda j: (j, 0, 0)),
        ],
        out_shape=[
            jax.ShapeDtypeStruct((NQ, NPAD), jnp.float32),
            jax.ShapeDtypeStruct((NPAD // KB, NQ, KB // CHUNK), jnp.float32),
        ],
    )(queries, keys_p)


def _sel_body(m_ref, sel_ref, buf):
    buf[...] = m_ref[...]
    sel_ref[...] = jnp.zeros_like(sel_ref)
    cols = lax.broadcasted_iota(jnp.int32, (NQ, NCHUNK), 1)
    rows128 = lax.broadcasted_iota(jnp.int32, (128, NQ), 0)

    def body(i, _):
        b = buf[...]
        mx = jnp.max(b, axis=1)
        eq = b == mx[:, None]
        idx = jnp.min(jnp.where(eq, cols, BIGI), axis=1)
        eq2 = eq & (cols == idx[:, None])
        buf[...] = jnp.where(eq2, NEG, b)
        sel_ref[...] = jnp.where(rows128 == i, idx.reshape(1, NQ),
                                 sel_ref[...])
        return 0

    lax.fori_loop(0, TOPK, body, 0)


def _select(m):
    return pl.pallas_call(
        _sel_body,
        out_shape=jax.ShapeDtypeStruct((128, NQ), jnp.int32),
        scratch_shapes=[pltpu.VMEM((NQ, NCHUNK), jnp.float32)],
    )(m)


NW = 32
ROWS_TOTAL = NQ * TOPK
ROWS_PER_W = ROWS_TOTAL // NW
WIN = 128
NWIN = ROWS_PER_W // WIN


def _gather_body(table_hbm, idx_hbm, out_hbm, idx_v, rows_v, sem):
    wid = lax.axis_index("s") * 2 + lax.axis_index("c")
    base0 = wid * ROWS_PER_W

    def win(w, _):
        base = base0 + w * WIN
        pltpu.sync_copy(idx_hbm.at[pl.ds(base, WIN)], idx_v)
        pltpu.async_copy(table_hbm.at[idx_v], rows_v, sem).wait()
        pltpu.sync_copy(rows_v, out_hbm.at[pl.ds(base, WIN)])
        return 0

    lax.fori_loop(0, NWIN, win, 0)


def _gather(table, idx_flat):
    mesh = plsc.VectorSubcoreMesh(core_axis_name="c", subcore_axis_name="s")
    f = functools.partial(
        pl.kernel,
        out_type=jax.ShapeDtypeStruct((ROWS_TOTAL, CHUNK), jnp.float32),
        mesh=mesh,
        scratch_types=[
            pltpu.VMEM((WIN,), jnp.int32),
            pltpu.VMEM((WIN, CHUNK), jnp.float32),
            pltpu.SemaphoreType.DMA,
        ],
    )(_gather_body)
    return f(table, idx_flat)


QB4 = 128


def _final_body(c_ref, d_ref, vals_ref, docs_ref, cb):
    cb[...] = c_ref[...]
    vals_ref[...] = jnp.zeros_like(vals_ref)
    docs_ref[...] = jnp.zeros_like(docs_ref)
    rows128 = lax.broadcasted_iota(jnp.int32, (128, QB4), 0)

    def body(i, _):
        b = cb[...]
        d = d_ref[...]
        mx = jnp.max(b, axis=1)
        eq = b == mx[:, None]
        docv = jnp.min(jnp.where(eq, d, BIGI), axis=1)
        eq2 = eq & (d == docv[:, None])
        cb[...] = jnp.where(eq2, NEG, b)
        hit = rows128 == i
        vals_ref[...] = jnp.where(hit, mx.reshape(1, QB4), vals_ref[...])
        docs_ref[...] = jnp.where(hit, docv.reshape(1, QB4), docs_ref[...])
        return 0

    lax.fori_loop(0, TOPK, body, 0)


def _final(cand, cand_doc):
    return pl.pallas_call(
        _final_body,
        grid=(NQ // QB4,),
        in_specs=[
            pl.BlockSpec((QB4, NSEL), lambda qi: (qi, 0)),
            pl.BlockSpec((QB4, NSEL), lambda qi: (qi, 0)),
        ],
        out_specs=[
            pl.BlockSpec((128, QB4), lambda qi: (0, qi)),
            pl.BlockSpec((128, QB4), lambda qi: (0, qi)),
        ],
        out_shape=[
            jax.ShapeDtypeStruct((128, NQ), jnp.float32),
            jax.ShapeDtypeStruct((128, NQ), jnp.int32),
        ],
        scratch_shapes=[pltpu.VMEM((QB4, NSEL), jnp.float32)],
    )(cand, cand_doc)


def kernel(queries, keys, k):
    n = keys.shape[0]
    keys_p = jnp.pad(keys, ((0, NPAD - n), (0, 0)))
    logits, m3 = _matmul(queries, keys_p)
    m = m3.transpose(1, 0, 2).reshape(NQ, NCHUNK)

    sel = _select(m)
    sel_t = sel[:TOPK].T

    qid = jnp.arange(NQ, dtype=jnp.int32)[:, None]
    idx_flat = (qid * NCHUNK + sel_t).reshape(ROWS_TOTAL)
    table = logits.reshape(NQ * NCHUNK, CHUNK)
    cand = _gather(table, idx_flat).reshape(NQ, NSEL)

    lane = jnp.arange(CHUNK, dtype=jnp.int32)
    cand_doc = (sel_t * CHUNK)[:, :, None] + lane[None, None, :]
    cand_doc = cand_doc.reshape(NQ, NSEL)

    vals, docs = _final(cand, cand_doc)
    values = vals[:TOPK].T
    indices = docs[:TOPK].T
    return values, indices
